# baseline (speedup 1.0000x reference)
import functools
import jax
import jax.numpy as jnp
from jax import lax
from jax.experimental import pallas as pl
from jax.experimental.pallas import tpu as pltpu
from jax.experimental.pallas import tpu_sc as plsc

_LANES = 16


def _make_sc_kernel(bs, h, w, nf):
    hw = h * w
    C = 2 * nf
    NC, NS = 2, 16
    NW = NC * NS
    CH = C // NW

    mesh = plsc.VectorSubcoreMesh(core_axis_name="c", subcore_axis_name="s")

    @functools.partial(
        pl.kernel,
        out_type=jax.ShapeDtypeStruct((bs * C, hw), jnp.float32),
        mesh=mesh,
        scratch_types=[
            pltpu.VMEM((CH * w,), jnp.float32),
            pltpu.VMEM((CH, hw), jnp.float32),
            pltpu.SemaphoreType.DMA,
        ],
        compiler_params=pltpu.CompilerParams(needs_layout_passes=False),
    )
    def sc_kernel(pat_hbm, out_hbm, pat_v, block_v, sem):
        wid = lax.axis_index("s") * NC + lax.axis_index("c")
        c0 = wid * CH
        pltpu.sync_copy(pat_hbm.at[pl.ds(c0 * w, CH * w)], pat_v)
        is_row = lax.broadcast(jnp.int32(1), (_LANES,)) * jnp.where(
            c0 >= nf, jnp.int32(1), jnp.int32(0)
        )
        lane = lax.iota(jnp.int32, _LANES)

        def build(g, carry):
            k = g * _LANES + lane
            idx = jnp.where(is_row > 0, lax.div(k, w), lax.rem(k, w))
            for r in range(CH):
                vals = plsc.load_gather(pat_v, [idx + r * w])
                block_v[r, pl.ds(g * _LANES, _LANES)] = vals
            return carry

        lax.fori_loop(0, hw // _LANES, build, 0)

        copies = [
            pltpu.async_copy(block_v, out_hbm.at[pl.ds(b * C + c0, CH)], sem)
            for b in range(bs)
        ]
        for cp in copies:
            cp.wait()

    return sc_kernel


def kernel(mask, row_embed, col_embed):
    bs = mask.shape[0]
    h, w = mask.shape[-2:]
    nf = row_embed.shape[1]
    pat = jnp.concatenate([col_embed[:w].T, row_embed[:h].T], axis=0)
    out = _make_sc_kernel(bs, h, w, nf)(pat.reshape(-1))
    return out.reshape(bs, 2 * nf, h, w)
